# baseline (speedup 1.0000x reference)
import functools

import jax
import jax.numpy as jnp
from jax import lax
from jax.experimental import pallas as pl
from jax.experimental.pallas import tpu as pltpu
from jax.experimental.pallas import tpu_sc as plsc

NC = 2
NS = 16
L = 16
NW = NC * NS
ROWS = 128
COLS = 32768
RPW = ROWS // NW
U = 8
SLICES = COLS // L
STEPS = SLICES // U
I32_MAX = 2**31 - 1


def _row_argmax(row_ref, lane_iota):
    ninf = jnp.full((L,), -jnp.inf, jnp.float32)
    zero = jnp.zeros((L,), jnp.int32)

    def body(i, carry):
        maxs, steps = carry
        ib = jnp.broadcast_to(i, (L,)).astype(jnp.int32)
        new_maxs = []
        new_steps = []
        base = i * (U * L)
        for k in range(U):
            v = row_ref[pl.ds(base + k * L, L)]
            take = v > maxs[k]
            new_maxs.append(jnp.where(take, v, maxs[k]))
            new_steps.append(jnp.where(take, ib, steps[k]))
        return tuple(new_maxs), tuple(new_steps)

    maxs, steps = lax.fori_loop(
        0, STEPS, body, ((ninf,) * U, (zero,) * U), unroll=False
    )

    m = maxs[0]
    idx = steps[0] * (U * L) + lane_iota
    for k in range(1, U):
        idx_k = steps[k] * (U * L) + (k * L) + lane_iota
        take = (maxs[k] > m) | ((maxs[k] == m) & (idx_k < idx))
        m = jnp.where(take, maxs[k], m)
        idx = jnp.where(take, idx_k, idx)

    dnums = lax.GatherDimensionNumbers(
        offset_dims=(), collapsed_slice_dims=(0,), start_index_map=(0,)
    )

    def shuf(v, perm):
        return lax.gather(
            v, perm[:, None], dnums, slice_sizes=(1,),
            mode=lax.GatherScatterMode.PROMISE_IN_BOUNDS,
        )

    for sh in (1, 2, 4, 8):
        perm = lane_iota ^ sh
        m2 = shuf(m, perm)
        idx2 = shuf(idx, perm)
        take = (m2 > m) | ((m2 == m) & (idx2 < idx))
        m = jnp.where(take, m2, m)
        idx = jnp.where(take, idx2, idx)
    return idx


_mesh = plsc.VectorSubcoreMesh(core_axis_name="c", subcore_axis_name="s")


@functools.partial(
    pl.kernel,
    out_type=jax.ShapeDtypeStruct((NW, L), jnp.int32),
    mesh=_mesh,
    scratch_types=[
        pltpu.VMEM((2, COLS), jnp.float32),
        pltpu.VMEM((L,), jnp.int32),
        pltpu.SemaphoreType.DMA,
        pltpu.SemaphoreType.DMA,
    ],
)
def _argmax_sc(x_hbm, out_hbm, buf, res_v, sem0, sem1):
    wid = lax.axis_index("s") * NC + lax.axis_index("c")
    r0 = wid * RPW
    sems = (sem0, sem1)
    lane_iota = lax.iota(jnp.int32, L)

    copies = [
        pltpu.make_async_copy(x_hbm.at[r0 + j], buf.at[j % 2], sems[j % 2])
        for j in range(RPW)
    ]
    copies[0].start()
    res = jnp.zeros((L,), jnp.int32)
    for j in range(RPW):
        if j + 1 < RPW:
            copies[j + 1].start()
        copies[j].wait()
        gidx = _row_argmax(buf.at[j % 2], lane_iota)
        res = jnp.where(lane_iota == j, gidx, res)
    res_v[...] = res
    pltpu.sync_copy(res_v, out_hbm.at[wid])


@jax.jit
def kernel(x):
    out = _argmax_sc(x)
    return out[:, :RPW].reshape(ROWS)
